# Initial kernel scaffold; baseline (speedup 1.0000x reference)
import jax
import jax.numpy as jnp
from jax.experimental import pallas as pl


def kernel(pos, vel, acc):
    raise NotImplementedError("write your pallas kernel here")


# speedup vs baseline: 2.2134x; 2.2134x over previous
import functools

import jax
import jax.numpy as jnp
from jax.experimental import pallas as pl
from jax.experimental.pallas import tpu as pltpu

LANE = 128
_GY = -9.81


def _fused_body(pos_ref, acc_ref, w_ref, f_ref, c_ref, *, n_sub):
    f32 = jnp.float32
    P = pos_ref[...]
    A = acc_ref[...]
    D = P.shape[1]
    TB = P.shape[0]

    Pt = P.T.reshape(D, n_sub, LANE)
    At = A.T.reshape(D, n_sub, LANE)

    def p(i):
        return Pt[i]

    ea, eb, ec = p(0), p(1), p(2)
    sx, cx = jnp.sin(ea), jnp.cos(ea)
    sy, cy = jnp.sin(eb), jnp.cos(eb)
    sz, cz = jnp.sin(ec), jnp.cos(ec)
    r00 = cz * cy
    r01 = cz * sy * sx - sz * cx
    r02 = cz * sy * cx + sz * sx
    r10 = sz * cy
    r11 = sz * sy * sx + cz * cx
    r12 = sz * sy * cx - cz * sx
    r20 = -sy
    r21 = cy * sx
    r22 = cy * cx

    px, py, pz = p(3), p(4), p(5)

    cax = At[0]
    cay = At[1] - f32(_GY)
    caz = At[2]

    contact = [(p(6 + i) < f32(0.1)).astype(f32) for i in range(2)]
    s = contact[0] + contact[1]
    active = (s > f32(0.0)).astype(f32)
    inv_s = jnp.where(s > f32(0.0), f32(1.0) / jnp.maximum(s, f32(1.0)), f32(0.0))
    fax, fay, faz = cax * inv_s, cay * inv_s, caz * inv_s

    w_planes, f_planes, c_planes = [], [], []
    for i in range(2):
        ci = contact[i]
        fx, fy, fz = ci * fax, ci * fay, ci * faz

        f_planes += [r00 * fx + r10 * fy + r20 * fz,
                     r01 * fx + r11 * fy + r21 * fz,
                     r02 * fx + r12 * fy + r22 * fz]

        wcx, wcy, wcz = p(8 + 3 * i), p(9 + 3 * i), p(10 + 3 * i)
        dx, dy, dz = wcx - px, wcy - py, wcz - pz
        c_planes += [active * (r00 * dx + r10 * dy + r20 * dz),
                     active * (r01 * dx + r11 * dy + r21 * dz),
                     active * (r02 * dx + r12 * dy + r22 * dz)]

        mx = wcy * fz - wcz * fy
        my = wcz * fx - wcx * fz
        mz = wcx * fy - wcy * fx

        bfx = r00 * fx + r01 * fy + r02 * fz
        bfy = r10 * fx + r11 * fy + r12 * fz
        bfz = r20 * fx + r21 * fy + r22 * fz
        w_planes += [r00 * mx + r01 * my + r02 * mz + (py * bfz - pz * bfy),
                     r10 * mx + r11 * my + r12 * mz + (pz * bfx - px * bfz),
                     r20 * mx + r21 * my + r22 * mz + (px * bfy - py * bfx),
                     bfx, bfy, bfz]

    w_ref[...] = jnp.stack(w_planes).reshape(12, TB).T
    f_ref[...] = jnp.stack(f_planes).reshape(6, TB).T
    c_ref[...] = jnp.stack(c_planes).reshape(6, TB).T


@functools.partial(jax.jit, static_argnums=())
def _contact_call(pos, acc):
    B, T, D = pos.shape
    TB = 0
    for cand in (1024, 512, 2048, 256, 128):
        if T % cand == 0:
            TB = cand
            break
    T_pad = T
    if TB == 0:
        T_pad = -(-T // LANE) * LANE
        pad = ((0, 0), (0, T_pad - T), (0, 0))
        pos = jnp.pad(pos, pad)
        acc = jnp.pad(acc, pad)
        TB = LANE
        for cand in (1024, 512, 256):
            if T_pad % cand == 0:
                TB = cand
                break
    n_t = T_pad // TB
    n_sub = TB // LANE

    body = functools.partial(_fused_body, n_sub=n_sub)
    idx = lambda b, t: (b, t, 0)
    wrench, force, cop = pl.pallas_call(
        body,
        grid=(B, n_t),
        in_specs=[pl.BlockSpec((None, TB, D), idx),
                  pl.BlockSpec((None, TB, D), idx)],
        out_specs=(pl.BlockSpec((None, TB, 12), idx),
                   pl.BlockSpec((None, TB, 6), idx),
                   pl.BlockSpec((None, TB, 6), idx)),
        out_shape=(jax.ShapeDtypeStruct((B, T_pad, 12), jnp.float32),
                   jax.ShapeDtypeStruct((B, T_pad, 6), jnp.float32),
                   jax.ShapeDtypeStruct((B, T_pad, 6), jnp.float32)),
        compiler_params=pltpu.CompilerParams(
            dimension_semantics=("parallel", "parallel")),
    )(pos, acc)
    if T_pad != T:
        wrench, force, cop = (x[:, :T] for x in (wrench, force, cop))
    return wrench, force, cop


def kernel(pos, vel, acc):
    del vel
    B, T, D = pos.shape
    wrench, force, cop = _contact_call(pos.astype(jnp.float32),
                                       acc.astype(jnp.float32))
    zeros = lambda f: jnp.zeros((B, T, f), jnp.float32)
    return {
        "groundContactWrenchesInRootFrame": wrench,
        "groundContactForcesInRootFrame": force,
        "groundContactCenterOfPressureInRootFrame": cop,
        "groundContactTorquesInRootFrame": zeros(6),
        "residualWrenchInRootFrame": zeros(6),
        "contact": zeros(2),
        "comAccInRootFrame": zeros(3),
        "tau": zeros(D),
    }
